# Initial kernel scaffold; baseline (speedup 1.0000x reference)
import jax
import jax.numpy as jnp
from jax.experimental import pallas as pl


def kernel(node_feats, edge_index, esn_state, W_gcn, b_gcn, ln_w, ln_b, W1, b1, W2, b2):
    raise NotImplementedError("write your pallas kernel here")


# speedup vs baseline: 82.4324x; 82.4324x over previous
import functools

import jax
import jax.numpy as jnp
from jax import lax
from jax.experimental import pallas as pl
from jax.experimental.pallas import tpu as pltpu
from jax.experimental.pallas import tpu_sc as plsc

N = 50000
E = 800000
HID = 128
NP = 50176
NROWS = NP // 128
EP = 819200
EROWS = EP // 128
NTILES = 32
ROWS_PT = EROWS // NTILES
KB = 8
NBLK = ROWS_PT // KB
SL = NP // 16
HBLK = 56
HSTEPS = NROWS // HBLK

_mesh = plsc.VectorSubcoreMesh(core_axis_name="c", subcore_axis_name="s")
_sc_params = pltpu.CompilerParams(use_tc_tiling_on_sc=False)


def _deg_body(dst_ref, zeros_ref, ones_ref, out0_ref, out1_ref,
              deg_sp, idxbuf, ones_v):
    c = lax.axis_index("c")
    s = lax.axis_index("s")
    wid = c * 16 + s

    @pl.when(s == 0)
    def _():
        pltpu.sync_copy(zeros_ref, deg_sp)

    pltpu.sync_copy(ones_ref, ones_v)
    plsc.subcore_barrier()
    row0 = wid * ROWS_PT

    def blk(b, carry):
        pltpu.sync_copy(dst_ref.at[pl.ds(row0 + b * KB, KB)], idxbuf)
        for j in range(KB):
            pltpu.sync_copy(ones_v, deg_sp.at[idxbuf.at[j]], add=True)
        return carry

    lax.fori_loop(0, NBLK, blk, 0)
    plsc.subcore_barrier()

    @pl.when((s == 0) & (c == 0))
    def _():
        pltpu.sync_copy(deg_sp, out0_ref)

    @pl.when((s == 0) & (c == 1))
    def _():
        pltpu.sync_copy(deg_sp, out1_ref)


_deg_call = functools.partial(
    pl.kernel,
    out_type=[
        jax.ShapeDtypeStruct((NP,), jnp.float32),
        jax.ShapeDtypeStruct((NP,), jnp.float32),
    ],
    mesh=_mesh,
    compiler_params=_sc_params,
    scratch_types=[
        pltpu.VMEM_SHARED((NP,), jnp.float32),
        pltpu.VMEM((KB, 128), jnp.int32),
        pltpu.VMEM((128,), jnp.float32),
    ],
)(_deg_body)


def _agg_body(src_ref, dst_ref, g0_ref, g1_ref, zeros_ref,
              o00_ref, o01_ref, o10_ref, o11_ref,
              g0_sp, g1_sp, acc0_sp, acc1_sp,
              sbuf, dbuf, vals0, vals1, sem0, sem1):
    c = lax.axis_index("c")
    s = lax.axis_index("s")
    wid = c * 16 + s

    @pl.when(s == 0)
    def _():
        pltpu.sync_copy(zeros_ref, acc0_sp)
        pltpu.sync_copy(zeros_ref, acc1_sp)
        pltpu.sync_copy(g0_ref, g0_sp)
        pltpu.sync_copy(g1_ref, g1_sp)

    plsc.subcore_barrier()
    row0 = wid * ROWS_PT

    def blk(b, carry):
        pltpu.sync_copy(src_ref.at[pl.ds(row0 + b * KB, KB)], sbuf)
        pltpu.sync_copy(dst_ref.at[pl.ds(row0 + b * KB, KB)], dbuf)
        for j in range(KB):
            cp0 = pltpu.async_copy(g0_sp.at[sbuf.at[j]], vals0.at[j], sem0)
            cp1 = pltpu.async_copy(g1_sp.at[sbuf.at[j]], vals1.at[j], sem1)
            cp0.wait()
            cp1.wait()
            pltpu.sync_copy(vals0.at[j], acc0_sp.at[dbuf.at[j]], add=True)
            pltpu.sync_copy(vals1.at[j], acc1_sp.at[dbuf.at[j]], add=True)
        return carry

    lax.fori_loop(0, NBLK, blk, 0)
    plsc.subcore_barrier()

    @pl.when((s == 0) & (c == 0))
    def _():
        pltpu.sync_copy(acc0_sp, o00_ref)
        pltpu.sync_copy(acc1_sp, o01_ref)

    @pl.when((s == 0) & (c == 1))
    def _():
        pltpu.sync_copy(acc0_sp, o10_ref)
        pltpu.sync_copy(acc1_sp, o11_ref)


_agg_call = functools.partial(
    pl.kernel,
    out_type=[
        jax.ShapeDtypeStruct((NP,), jnp.float32),
        jax.ShapeDtypeStruct((NP,), jnp.float32),
        jax.ShapeDtypeStruct((NP,), jnp.float32),
        jax.ShapeDtypeStruct((NP,), jnp.float32),
    ],
    mesh=_mesh,
    compiler_params=_sc_params,
    scratch_types=[
        pltpu.VMEM_SHARED((NP,), jnp.float32),
        pltpu.VMEM_SHARED((NP,), jnp.float32),
        pltpu.VMEM_SHARED((NP,), jnp.float32),
        pltpu.VMEM_SHARED((NP,), jnp.float32),
        pltpu.VMEM((KB, 128), jnp.int32),
        pltpu.VMEM((KB, 128), jnp.int32),
        pltpu.VMEM((KB, 128), jnp.float32),
        pltpu.VMEM((KB, 128), jnp.float32),
        pltpu.SemaphoreType.DMA,
        pltpu.SemaphoreType.DMA,
    ],
)(_agg_body)


def _prep_body(degp_ref, f0_ref, f1_ref, dinv_ref, g0_ref, g1_ref):
    deg = degp_ref[0] + degp_ref[1] + 1.0
    dv = lax.rsqrt(deg)
    dinv_ref[...] = dv
    g0_ref[...] = f0_ref[...] * dv
    g1_ref[...] = f1_ref[...] * dv


_prep_call = pl.pallas_call(
    _prep_body,
    out_shape=[
        jax.ShapeDtypeStruct((NROWS, 128), jnp.float32),
        jax.ShapeDtypeStruct((NROWS, 128), jnp.float32),
        jax.ShapeDtypeStruct((NROWS, 128), jnp.float32),
    ],
)


def _head_body(ac0_ref, ac1_ref, dinv_ref, g0_ref, g1_ref, wg_ref, bg_ref,
               lnw_ref, lnb_ref, esn_ref, w1_ref, b1_ref, w2_ref, b2_ref,
               out_ref, ssum, ssq):
    i = pl.program_id(0)

    @pl.when(i == 0)
    def _():
        ssum[...] = jnp.zeros((1, HID), jnp.float32)
        ssq[...] = jnp.zeros((1, HID), jnp.float32)

    a0 = dinv_ref[...] * (ac0_ref[0] + ac0_ref[1] + g0_ref[...])
    a1 = dinv_ref[...] * (ac1_ref[0] + ac1_ref[1] + g1_ref[...])
    wg = wg_ref[...]
    bg = bg_ref[...]
    x = (a0[:, :, None] * wg[0][None, None, :]
         + a1[:, :, None] * wg[1][None, None, :]
         + bg[0][None, None, :])
    x = jnp.maximum(x, 0.0)
    rows = lax.broadcasted_iota(jnp.int32, (HBLK, 128), 0)
    cols = lax.broadcasted_iota(jnp.int32, (HBLK, 128), 1)
    node = (i * HBLK + rows) * 128 + cols
    m = (node < N).astype(jnp.float32)
    x = x * m[:, :, None]
    ssum[...] += x.sum((0, 1))[None, :]
    ssq[...] += (x * x).sum((0, 1))[None, :]

    @pl.when(i == HSTEPS - 1)
    def _():
        s_ch = ssum[...]
        s1 = jnp.sum(s_ch)
        s2 = jnp.sum(ssq[...])
        cnt = float(N) * float(HID)
        mean = s1 / cnt
        std = jnp.sqrt(s2 / cnt - mean * mean)
        pooled = ((s_ch - float(N) * mean) / (std + 1e-5) * lnw_ref[...]
                  + float(N) * lnb_ref[...])
        z = jnp.concatenate([pooled, esn_ref[...]], axis=1)
        z1 = jnp.dot(z, w1_ref[...], preferred_element_type=jnp.float32)
        z1 = jnp.maximum(z1 + b1_ref[...], 0.0)
        lg = jnp.dot(z1, w2_ref[...], preferred_element_type=jnp.float32)
        lg = lg + b2_ref[...]
        mx = jnp.max(lg, axis=1, keepdims=True)
        out_ref[...] = lg - mx - jnp.log(
            jnp.sum(jnp.exp(lg - mx), axis=1, keepdims=True))


_head_call = pl.pallas_call(
    _head_body,
    grid=(HSTEPS,),
    in_specs=[
        pl.BlockSpec((2, HBLK, 128), lambda i: (0, i, 0)),
        pl.BlockSpec((2, HBLK, 128), lambda i: (0, i, 0)),
        pl.BlockSpec((HBLK, 128), lambda i: (i, 0)),
        pl.BlockSpec((HBLK, 128), lambda i: (i, 0)),
        pl.BlockSpec((HBLK, 128), lambda i: (i, 0)),
        pl.BlockSpec((2, 128), lambda i: (0, 0)),
        pl.BlockSpec((1, 128), lambda i: (0, 0)),
        pl.BlockSpec((1, 128), lambda i: (0, 0)),
        pl.BlockSpec((1, 128), lambda i: (0, 0)),
        pl.BlockSpec((1, 500), lambda i: (0, 0)),
        pl.BlockSpec((HID + 500, HID), lambda i: (0, 0)),
        pl.BlockSpec((1, 128), lambda i: (0, 0)),
        pl.BlockSpec((HID, 64), lambda i: (0, 0)),
        pl.BlockSpec((1, 64), lambda i: (0, 0)),
    ],
    out_specs=pl.BlockSpec((1, 64), lambda i: (0, 0)),
    out_shape=jax.ShapeDtypeStruct((1, 64), jnp.float32),
    scratch_shapes=[
        pltpu.VMEM((1, HID), jnp.float32),
        pltpu.VMEM((1, HID), jnp.float32),
    ],
)


def kernel(node_feats, edge_index, esn_state, W_gcn, b_gcn, ln_w, ln_b,
           W1, b1, W2, b2):
    src = edge_index[0]
    dst = edge_index[1]
    pad = (N + (jnp.arange(EP - E, dtype=jnp.int32) % (NP - N))).astype(jnp.int32)
    srcp = jnp.concatenate([src, pad]).reshape(EROWS, 128)
    dstp = jnp.concatenate([dst, pad]).reshape(EROWS, 128)
    zeros1 = jnp.zeros((NP,), jnp.float32)
    ones = jnp.ones((128,), jnp.float32)

    deg0, deg1 = _deg_call(dstp, zeros1, ones)
    degp = jnp.stack([deg0, deg1]).reshape(2, NROWS, 128)

    fpad = jnp.pad(node_feats, ((0, NP - N), (0, 0)))
    f0 = fpad[:, 0].reshape(NROWS, 128)
    f1 = fpad[:, 1].reshape(NROWS, 128)
    dinv, g0, g1 = _prep_call(degp, f0, f1)

    o00, o01, o10, o11 = _agg_call(srcp, dstp, g0.reshape(-1),
                                   g1.reshape(-1), zeros1)

    ac0 = jnp.stack([o00, o10]).reshape(2, NROWS, 128)
    ac1 = jnp.stack([o01, o11]).reshape(2, NROWS, 128)
    out = _head_call(ac0, ac1, dinv, g0, g1, W_gcn,
                     b_gcn.reshape(1, 128), ln_w.reshape(1, 128),
                     ln_b.reshape(1, 128), esn_state, W1,
                     b1.reshape(1, 128), W2, b2.reshape(1, 64))
    return out


# speedup vs baseline: 180.5410x; 2.1902x over previous
import functools

import jax
import jax.numpy as jnp
from jax import lax
from jax.experimental import pallas as pl
from jax.experimental.pallas import tpu as pltpu
from jax.experimental.pallas import tpu_sc as plsc

N = 50000
E = 800000
HID = 128
NP = 50176
NROWS = NP // 128
NTILES = 32
EPT = E // NTILES
HBLK = 56
HSTEPS = NROWS // HBLK

_mesh = plsc.VectorSubcoreMesh(core_axis_name="c", subcore_axis_name="s")
_sc_params = pltpu.CompilerParams(use_tc_tiling_on_sc=False)


def _deg_body(ei_ref, zeros_ref, ones_ref, out0_ref, out1_ref,
              deg_sp, dbuf, ones_v):
    c = lax.axis_index("c")
    s = lax.axis_index("s")
    wid = c * 16 + s

    @pl.when(s == 0)
    def _():
        pltpu.sync_copy(zeros_ref, deg_sp)

    pltpu.sync_copy(ones_ref, ones_v)
    pltpu.sync_copy(ei_ref.at[1, pl.ds(wid * EPT, EPT)], dbuf)
    plsc.subcore_barrier()
    pltpu.sync_copy(ones_v, deg_sp.at[dbuf], add=True)
    plsc.subcore_barrier()

    @pl.when((s == 0) & (c == 0))
    def _():
        pltpu.sync_copy(deg_sp, out0_ref)

    @pl.when((s == 0) & (c == 1))
    def _():
        pltpu.sync_copy(deg_sp, out1_ref)


_deg_call = functools.partial(
    pl.kernel,
    out_type=[
        jax.ShapeDtypeStruct((NP,), jnp.float32),
        jax.ShapeDtypeStruct((NP,), jnp.float32),
    ],
    mesh=_mesh,
    compiler_params=_sc_params,
    scratch_types=[
        pltpu.VMEM_SHARED((NP,), jnp.float32),
        pltpu.VMEM((EPT,), jnp.int32),
        pltpu.VMEM((EPT,), jnp.float32),
    ],
)(_deg_body)


def _agg_body(ei_ref, g0_ref, g1_ref, zeros_ref,
              o00_ref, o01_ref, o10_ref, o11_ref,
              g0_sp, g1_sp, acc0_sp, acc1_sp,
              sbuf, dbuf, vals0, vals1, sem0, sem1):
    c = lax.axis_index("c")
    s = lax.axis_index("s")
    wid = c * 16 + s

    @pl.when(s == 0)
    def _():
        pltpu.sync_copy(zeros_ref, acc0_sp)
        pltpu.sync_copy(zeros_ref, acc1_sp)
        pltpu.sync_copy(g0_ref, g0_sp)
        pltpu.sync_copy(g1_ref, g1_sp)

    base = wid * EPT
    cps = pltpu.async_copy(ei_ref.at[0, pl.ds(base, EPT)], sbuf, sem0)
    cpd = pltpu.async_copy(ei_ref.at[1, pl.ds(base, EPT)], dbuf, sem1)
    cps.wait()
    cpd.wait()
    plsc.subcore_barrier()
    cp0 = pltpu.async_copy(g0_sp.at[sbuf], vals0, sem0)
    cp1 = pltpu.async_copy(g1_sp.at[sbuf], vals1, sem1)
    cp0.wait()
    pltpu.sync_copy(vals0, acc0_sp.at[dbuf], add=True)
    cp1.wait()
    pltpu.sync_copy(vals1, acc1_sp.at[dbuf], add=True)
    plsc.subcore_barrier()

    @pl.when((s == 0) & (c == 0))
    def _():
        pltpu.sync_copy(acc0_sp, o00_ref)
        pltpu.sync_copy(acc1_sp, o01_ref)

    @pl.when((s == 0) & (c == 1))
    def _():
        pltpu.sync_copy(acc0_sp, o10_ref)
        pltpu.sync_copy(acc1_sp, o11_ref)


_agg_call = functools.partial(
    pl.kernel,
    out_type=[
        jax.ShapeDtypeStruct((NP,), jnp.float32),
        jax.ShapeDtypeStruct((NP,), jnp.float32),
        jax.ShapeDtypeStruct((NP,), jnp.float32),
        jax.ShapeDtypeStruct((NP,), jnp.float32),
    ],
    mesh=_mesh,
    compiler_params=_sc_params,
    scratch_types=[
        pltpu.VMEM_SHARED((NP,), jnp.float32),
        pltpu.VMEM_SHARED((NP,), jnp.float32),
        pltpu.VMEM_SHARED((NP,), jnp.float32),
        pltpu.VMEM_SHARED((NP,), jnp.float32),
        pltpu.VMEM((EPT,), jnp.int32),
        pltpu.VMEM((EPT,), jnp.int32),
        pltpu.VMEM((EPT,), jnp.float32),
        pltpu.VMEM((EPT,), jnp.float32),
        pltpu.SemaphoreType.DMA,
        pltpu.SemaphoreType.DMA,
    ],
)(_agg_body)


def _prep_body(d0_ref, d1_ref, f0_ref, f1_ref, dinv_ref, g0_ref, g1_ref):
    deg = d0_ref[...] + d1_ref[...] + 1.0
    dv = lax.rsqrt(deg)
    dinv_ref[...] = dv
    g0_ref[...] = f0_ref[...] * dv
    g1_ref[...] = f1_ref[...] * dv


_prep_call = pl.pallas_call(
    _prep_body,
    out_shape=[
        jax.ShapeDtypeStruct((NROWS, 128), jnp.float32),
        jax.ShapeDtypeStruct((NROWS, 128), jnp.float32),
        jax.ShapeDtypeStruct((NROWS, 128), jnp.float32),
    ],
)


def _head_body(a00_ref, a10_ref, a01_ref, a11_ref,
               dinv_ref, g0_ref, g1_ref, wg_ref, bg_ref,
               lnw_ref, lnb_ref, esn_ref, w1_ref, b1_ref, w2_ref, b2_ref,
               out_ref, ssum, ssq):
    i = pl.program_id(0)

    @pl.when(i == 0)
    def _():
        ssum[...] = jnp.zeros((1, HID), jnp.float32)
        ssq[...] = jnp.zeros((1, HID), jnp.float32)

    a0 = dinv_ref[...] * (a00_ref[...] + a10_ref[...] + g0_ref[...])
    a1 = dinv_ref[...] * (a01_ref[...] + a11_ref[...] + g1_ref[...])
    wg = wg_ref[...]
    bg = bg_ref[...]
    x = (a0[:, :, None] * wg[0][None, None, :]
         + a1[:, :, None] * wg[1][None, None, :]
         + bg[0][None, None, :])
    x = jnp.maximum(x, 0.0)
    rows = lax.broadcasted_iota(jnp.int32, (HBLK, 128), 0)
    cols = lax.broadcasted_iota(jnp.int32, (HBLK, 128), 1)
    node = (i * HBLK + rows) * 128 + cols
    m = (node < N).astype(jnp.float32)
    x = x * m[:, :, None]
    ssum[...] += x.sum((0, 1))[None, :]
    ssq[...] += (x * x).sum((0, 1))[None, :]

    @pl.when(i == HSTEPS - 1)
    def _():
        s_ch = ssum[...]
        s1 = jnp.sum(s_ch)
        s2 = jnp.sum(ssq[...])
        cnt = float(N) * float(HID)
        mean = s1 / cnt
        std = jnp.sqrt(s2 / cnt - mean * mean)
        pooled = ((s_ch - float(N) * mean) / (std + 1e-5) * lnw_ref[...]
                  + float(N) * lnb_ref[...])
        z = jnp.concatenate([pooled, esn_ref[...]], axis=1)
        z1 = jnp.dot(z, w1_ref[...], preferred_element_type=jnp.float32)
        z1 = jnp.maximum(z1 + b1_ref[...], 0.0)
        lg = jnp.dot(z1, w2_ref[...], preferred_element_type=jnp.float32)
        lg = lg + b2_ref[...]
        mx = jnp.max(lg, axis=1, keepdims=True)
        out_ref[...] = lg - mx - jnp.log(
            jnp.sum(jnp.exp(lg - mx), axis=1, keepdims=True))


_blk = pl.BlockSpec((HBLK, 128), lambda i: (i, 0))
_whole = lambda shape: pl.BlockSpec(shape, lambda i: tuple(0 for _ in shape))

_head_call = pl.pallas_call(
    _head_body,
    grid=(HSTEPS,),
    in_specs=[
        _blk, _blk, _blk, _blk, _blk, _blk, _blk,
        _whole((2, 128)),
        _whole((1, 128)),
        _whole((1, 128)),
        _whole((1, 128)),
        _whole((1, 500)),
        _whole((HID + 500, HID)),
        _whole((1, 128)),
        _whole((HID, 64)),
        _whole((1, 64)),
    ],
    out_specs=pl.BlockSpec((1, 64), lambda i: (0, 0)),
    out_shape=jax.ShapeDtypeStruct((1, 64), jnp.float32),
    scratch_shapes=[
        pltpu.VMEM((1, HID), jnp.float32),
        pltpu.VMEM((1, HID), jnp.float32),
    ],
)


def kernel(node_feats, edge_index, esn_state, W_gcn, b_gcn, ln_w, ln_b,
           W1, b1, W2, b2):
    zeros1 = jnp.zeros((NP,), jnp.float32)
    ones_e = jnp.ones((EPT,), jnp.float32)

    deg0, deg1 = _deg_call(edge_index, zeros1, ones_e)

    fpad = jnp.pad(node_feats, ((0, NP - N), (0, 0)))
    f0 = fpad[:, 0].reshape(NROWS, 128)
    f1 = fpad[:, 1].reshape(NROWS, 128)
    dinv, g0, g1 = _prep_call(deg0.reshape(NROWS, 128),
                              deg1.reshape(NROWS, 128), f0, f1)

    o00, o01, o10, o11 = _agg_call(edge_index, g0.reshape(-1),
                                   g1.reshape(-1), zeros1)

    out = _head_call(o00.reshape(NROWS, 128), o10.reshape(NROWS, 128),
                     o01.reshape(NROWS, 128), o11.reshape(NROWS, 128),
                     dinv, g0, g1, W_gcn,
                     b_gcn.reshape(1, 128), ln_w.reshape(1, 128),
                     ln_b.reshape(1, 128), esn_state, W1,
                     b1.reshape(1, 128), W2, b2.reshape(1, 64))
    return out
